# baseline (speedup 1.0000x reference)
import functools

import jax
import jax.numpy as jnp
from jax import lax
from jax.experimental import pallas as pl
from jax.experimental.pallas import tpu as pltpu
from jax.experimental.pallas import tpu_sc as plsc


@functools.partial(jax.jit, static_argnames=("n", "per_w", "sub", "iters"))
def _lookup_flat(x_flat, w_flat, *, n, per_w, sub, iters):
    mesh = plsc.VectorSubcoreMesh(core_axis_name="c", subcore_axis_name="s")
    info = plsc.get_sparse_core_info()
    nc = info.num_cores
    n_grp = sub // 16

    @functools.partial(
        pl.kernel,
        mesh=mesh,
        out_type=jax.ShapeDtypeStruct((4 * n,), jnp.float32),
        scratch_types=[
            pltpu.VMEM((sub,), jnp.int32),
            pltpu.VMEM((sub,), jnp.int32),
            pltpu.VMEM((4 * sub,), jnp.float32),
            pltpu.VMEM((4 * sub,), jnp.float32),
            pltpu.VMEM((16,), jnp.float32),
            pltpu.SemaphoreType.DMA,
            pltpu.SemaphoreType.DMA,
            pltpu.SemaphoreType.DMA,
        ],
        compiler_params=pltpu.CompilerParams(needs_layout_passes=False),
    )
    def k(
        x_hbm,
        w_hbm,
        out_hbm,
        x_v0,
        x_v1,
        out_v0,
        out_v1,
        tbl_v,
        in_sem,
        out_sem0,
        out_sem1,
    ):
        wid = lax.axis_index("s") * nc + lax.axis_index("c")
        base = wid * per_w

        pltpu.sync_copy(w_hbm, tbl_v)
        io = lax.iota(jnp.int32, 16)
        r = lax.bitwise_and(io, 3)
        q = lax.shift_right_logical(io, 2)
        perms = [q + 4 * m for m in range(4)]
        x_vs = (x_v0, x_v1)
        out_vs = (out_v0, out_v1)
        out_sems = (out_sem0, out_sem1)

        def start_in(it, slot):
            pltpu.async_copy(
                x_hbm.at[pl.ds(base + it * sub, sub)], x_vs[slot], in_sem
            )

        def wait_in(slot):
            pltpu.make_async_copy(
                x_hbm.at[pl.ds(0, sub)], x_vs[slot], in_sem
            ).wait()

        def wait_out(slot):
            pltpu.make_async_copy(
                out_vs[slot], out_hbm.at[pl.ds(0, 4 * sub)], out_sems[slot]
            ).wait()

        start_in(0, 0)

        def pair_body(it2, _):
            for slot in (0, 1):
                it = 2 * it2 + slot
                x_v = x_vs[slot]
                out_v = out_vs[slot]
                wait_in(slot)

                @pl.when(it + 1 < iters)
                def _():
                    start_in(it + 1, 1 - slot)

                @pl.when(it2 >= 1)
                def _():
                    wait_out(slot)

                def grp(g, _):
                    off = g * 16
                    for m in range(4):
                        xg = plsc.load_gather(x_v, [off + perms[m]])
                        ti = lax.shift_left(xg, 2) + r
                        o = plsc.load_gather(tbl_v, [ti])
                        out_v[pl.ds(g * 64 + m * 16, 16)] = o
                    return 0

                lax.fori_loop(0, n_grp, grp, 0, unroll=8)

                pltpu.async_copy(
                    out_v,
                    out_hbm.at[pl.ds(4 * (base + it * sub), 4 * sub)],
                    out_sems[slot],
                )
            return 0

        lax.fori_loop(0, iters // 2, pair_body, 0)
        wait_out(0)
        wait_out(1)

    return k(x_flat, w_flat)


def kernel(x, emb_weight):
    b, t = x.shape
    n = b * t
    x_flat = x.reshape(n).astype(jnp.int32)
    w_flat = emb_weight.reshape(16).astype(jnp.float32)

    nw = 32
    per_w = n // nw
    assert per_w * nw == n
    sub = max(
        c
        for c in range(16, 12801, 16)
        if per_w % c == 0 and (per_w // c) % 2 == 0
    )
    iters = per_w // sub

    y_flat = _lookup_flat(x_flat, w_flat, n=n, per_w=per_w, sub=sub, iters=iters)
    return y_flat.reshape(b, t, 4)


# speedup vs baseline: 19.3300x; 19.3300x over previous
import functools

import jax
import jax.numpy as jnp
from jax import lax
from jax.experimental import pallas as pl
from jax.experimental.pallas import tpu as pltpu
from jax.experimental.pallas import tpu_sc as plsc


@functools.partial(jax.jit, static_argnames=("n", "blk_w", "blk_c", "iters"))
def _lookup_t(x_t, w2, *, n, blk_w, blk_c, iters):
    mesh = plsc.VectorSubcoreMesh(core_axis_name="c", subcore_axis_name="s")
    info = plsc.get_sparse_core_info()
    nc = info.num_cores
    sub_x = blk_c * 128
    sub_o = blk_c * 512

    @functools.partial(
        pl.kernel,
        mesh=mesh,
        out_type=jax.ShapeDtypeStruct((4 * n,), jnp.float32),
        scratch_types=[
            pltpu.VMEM((sub_x,), jnp.int32),
            pltpu.VMEM((sub_x,), jnp.int32),
            pltpu.VMEM((sub_o,), jnp.float32),
            pltpu.VMEM((sub_o,), jnp.float32),
            pltpu.VMEM((16, 16), jnp.float32),
            pltpu.SemaphoreType.DMA,
            pltpu.SemaphoreType.DMA,
            pltpu.SemaphoreType.DMA,
        ],
        compiler_params=pltpu.CompilerParams(needs_layout_passes=False),
    )
    def k(
        x_hbm,
        w_hbm,
        out_hbm,
        x_v0,
        x_v1,
        out_v0,
        out_v1,
        tbl_v,
        in_sem,
        out_sem0,
        out_sem1,
    ):
        wid = lax.axis_index("s") * nc + lax.axis_index("c")
        xbase = wid * blk_w * 128
        obase = wid * blk_w * 512

        pltpu.sync_copy(w_hbm, tbl_v)
        io = lax.iota(jnp.int32, 16)
        x_vs = (x_v0, x_v1)
        out_vs = (out_v0, out_v1)
        out_sems = (out_sem0, out_sem1)

        def start_in(it, slot):
            off = pl.multiple_of(xbase + it * sub_x, 8)
            pltpu.async_copy(x_hbm.at[pl.ds(off, sub_x)], x_vs[slot], in_sem)

        def wait_in(slot):
            pltpu.make_async_copy(
                x_hbm.at[pl.ds(0, sub_x)], x_vs[slot], in_sem
            ).wait()

        def wait_out(slot):
            pltpu.make_async_copy(
                out_vs[slot], out_hbm.at[pl.ds(0, sub_o)], out_sems[slot]
            ).wait()

        start_in(0, 0)

        def pair_body(it2, _):
            for slot in (0, 1):
                it = 2 * it2 + slot
                x_v = x_vs[slot]
                out_v = out_vs[slot]
                wait_in(slot)

                @pl.when(it + 1 < iters)
                def _():
                    start_in(it + 1, 1 - slot)

                @pl.when(it2 >= 1)
                def _():
                    wait_out(slot)

                def blk_body(blk, _):
                    xb = blk * 128
                    ob = blk * 512
                    tix = [
                        lax.shift_left(x_v[pl.ds(xb + 16 * v, 16)], 2)
                        for v in range(8)
                    ]
                    for kk in range(4):
                        for v in range(8):
                            o = plsc.load_gather(tbl_v, [tix[v] + kk, io])
                            out_v[pl.ds(ob + 128 * kk + 16 * v, 16)] = o
                    return 0

                lax.fori_loop(0, blk_c, blk_body, 0, unroll=2)

                pltpu.async_copy(
                    out_v,
                    out_hbm.at[pl.ds(obase + it * sub_o, sub_o)],
                    out_sems[slot],
                )
            return 0

        lax.fori_loop(0, iters // 2, pair_body, 0)
        wait_out(0)
        wait_out(1)

    return k(x_t, w2)


def kernel(x, emb_weight):
    b, t = x.shape
    n = b * t
    x_t = x.T.reshape(n).astype(jnp.int32)
    w_flat = emb_weight.reshape(16).astype(jnp.float32)
    w2 = jnp.tile(w_flat[:, None], (1, 16))

    nw = 32
    n_blk = n // 128
    blk_w = n_blk // nw
    assert blk_w * nw == n_blk and (b % 128) == 0
    blk_c = max(
        c for c in range(2, 81, 2) if blk_w % c == 0 and (blk_w // c) % 2 == 0
    )
    iters = blk_w // blk_c

    y_flat = _lookup_t(x_t, w2, n=n, blk_w=blk_w, blk_c=blk_c, iters=iters)
    z = y_flat.reshape(t, b // 128, 4, 128)
    return z.transpose(1, 3, 0, 2).reshape(b, t, 4)


# speedup vs baseline: 37.1562x; 1.9222x over previous
import functools

import jax
import jax.numpy as jnp
from jax import lax
from jax.experimental import pallas as pl
from jax.experimental.pallas import tpu as pltpu
from jax.experimental.pallas import tpu_sc as plsc


@functools.partial(jax.jit, static_argnames=("n", "blk_w", "blk_c", "iters"))
def _lookup_t(x_t, w2, *, n, blk_w, blk_c, iters):
    mesh = plsc.VectorSubcoreMesh(core_axis_name="c", subcore_axis_name="s")
    info = plsc.get_sparse_core_info()
    nc = info.num_cores
    sub_x = blk_c * 128
    sub_o = blk_c * 512

    @functools.partial(
        pl.kernel,
        mesh=mesh,
        out_type=jax.ShapeDtypeStruct((4 * n,), jnp.float32),
        scratch_types=[
            pltpu.VMEM((sub_x,), jnp.int32),
            pltpu.VMEM((sub_x,), jnp.int32),
            pltpu.VMEM((sub_o,), jnp.float32),
            pltpu.VMEM((sub_o,), jnp.float32),
            pltpu.VMEM((16, 16), jnp.float32),
            pltpu.SemaphoreType.DMA,
            pltpu.SemaphoreType.DMA,
            pltpu.SemaphoreType.DMA,
        ],
        compiler_params=pltpu.CompilerParams(needs_layout_passes=False),
    )
    def k(
        x_hbm,
        w_hbm,
        out_hbm,
        x_v0,
        x_v1,
        out_v0,
        out_v1,
        tbl_v,
        in_sem,
        out_sem0,
        out_sem1,
    ):
        wid = lax.axis_index("s") * nc + lax.axis_index("c")
        xbase = wid * blk_w * 128
        obase = wid * blk_w * 512

        pltpu.sync_copy(w_hbm, tbl_v)
        io = lax.iota(jnp.int32, 16)
        x_vs = (x_v0, x_v1)
        out_vs = (out_v0, out_v1)
        out_sems = (out_sem0, out_sem1)

        def start_in(it, slot):
            off = pl.multiple_of(xbase + it * sub_x, 8)
            pltpu.async_copy(x_hbm.at[pl.ds(off, sub_x)], x_vs[slot], in_sem)

        def wait_in(slot):
            pltpu.make_async_copy(
                x_hbm.at[pl.ds(0, sub_x)], x_vs[slot], in_sem
            ).wait()

        def wait_out(slot):
            pltpu.make_async_copy(
                out_vs[slot], out_hbm.at[pl.ds(0, sub_o)], out_sems[slot]
            ).wait()

        start_in(0, 0)

        def pair_body(it2, _):
            for slot in (0, 1):
                it = 2 * it2 + slot
                x_v = x_vs[slot]
                out_v = out_vs[slot]
                wait_in(slot)

                @pl.when(it + 1 < iters)
                def _():
                    start_in(it + 1, 1 - slot)

                @pl.when(it2 >= 1)
                def _():
                    wait_out(slot)

                @plsc.parallel_loop(0, blk_c, unroll=2)
                def blk_body(blk):
                    xb = blk * 128
                    ob = blk * 512
                    tix = [
                        lax.shift_left(x_v[pl.ds(xb + 16 * v, 16)], 2)
                        for v in range(8)
                    ]
                    for kk in range(4):
                        os = [
                            plsc.load_gather(tbl_v, [tix[v] + kk, io])
                            for v in range(8)
                        ]
                        for v in range(8):
                            out_v[pl.ds(ob + 128 * kk + 16 * v, 16)] = os[v]

                pltpu.async_copy(
                    out_v,
                    out_hbm.at[pl.ds(obase + it * sub_o, sub_o)],
                    out_sems[slot],
                )
            return 0

        lax.fori_loop(0, iters // 2, pair_body, 0)
        wait_out(0)
        wait_out(1)

    return k(x_t, w2)


def kernel(x, emb_weight):
    b, t = x.shape
    n = b * t
    x_t = x.T.reshape(n).astype(jnp.int32)
    w_flat = emb_weight.reshape(16).astype(jnp.float32)
    w2 = jnp.tile(w_flat[:, None], (1, 16))

    nw = 32
    n_blk = n // 128
    blk_w = n_blk // nw
    assert blk_w * nw == n_blk and (b % 128) == 0
    blk_c = max(
        c for c in range(2, 81, 2) if blk_w % c == 0 and (blk_w // c) % 2 == 0
    )
    iters = blk_w // blk_c

    y_flat = _lookup_t(x_t, w2, n=n, blk_w=blk_w, blk_c=blk_c, iters=iters)
    z = y_flat.reshape(t, b // 128, 4, 128)
    return z.transpose(1, 3, 0, 2).reshape(b, t, 4)
